# baseline (speedup 1.0000x reference)
import functools

import jax
import jax.numpy as jnp
from jax import lax
from jax.experimental import pallas as pl
from jax.experimental.pallas import tpu as pltpu
from jax.experimental.pallas import tpu_sc as plsc

B, H, E, O = 16384, 50, 32, 64
N = B * H
NC, NS = 2, 16
NW = NC * NS
PER_W = N // NW
GCHUNK = 128
CHUNK = 1024
NG = CHUNK // GCHUNK
NCHUNKS = PER_W // CHUNK


def _sc_gather(idx2d, table):
    mesh = plsc.VectorSubcoreMesh(core_axis_name="c", subcore_axis_name="s")

    @functools.partial(
        pl.kernel,
        mesh=mesh,
        out_type=jax.ShapeDtypeStruct((N, E), jnp.float32),
        scratch_types=[
            pltpu.VMEM((NG, GCHUNK), jnp.int32),
            pltpu.VMEM((CHUNK, E), jnp.float32),
            pltpu.SemaphoreType.DMA,
        ],
        compiler_params=pltpu.CompilerParams(use_tc_tiling_on_sc=False),
    )
    def k(idx_hbm, table_hbm, out_hbm, idx_v, rows_v, sem):
        wid = lax.axis_index("s") * NC + lax.axis_index("c")
        base = wid * PER_W

        def body(i, carry):
            off = pl.multiple_of(base + i * CHUNK, CHUNK)
            pltpu.sync_copy(idx_hbm.at[pl.ds(pl.multiple_of(off // GCHUNK, NG), NG)], idx_v)
            copies = [
                pltpu.async_copy(
                    table_hbm.at[idx_v.at[j]],
                    rows_v.at[pl.ds(j * GCHUNK, GCHUNK)],
                    sem,
                )
                for j in range(NG)
            ]
            for cp in copies:
                cp.wait()
            pltpu.sync_copy(rows_v, out_hbm.at[pl.ds(off, CHUNK)])
            return carry

        lax.fori_loop(0, NCHUNKS, body, 0)

    return k(idx2d, table)


BB = 128
RB = BB * H


def _tc_linear_relu(xhbm, wt, b2):
    def body(x_hbm, w_ref, b_ref, o_ref, x_vmem, sem):
        i = pl.program_id(0)
        cp = pltpu.make_async_copy(x_hbm.at[pl.ds(i * RB, RB)], x_vmem, sem)
        cp.start()
        cp.wait()
        acc = jnp.dot(x_vmem[...], w_ref[...], preferred_element_type=jnp.float32)
        acc = jnp.maximum(acc + b_ref[...], 0.0)
        o_ref[...] = acc.reshape(BB, H, O)

    return pl.pallas_call(
        body,
        grid=(B // BB,),
        in_specs=[
            pl.BlockSpec(memory_space=pltpu.MemorySpace.HBM),
            pl.BlockSpec((E, O), lambda i: (0, 0)),
            pl.BlockSpec((1, O), lambda i: (0, 0)),
        ],
        out_specs=pl.BlockSpec((BB, H, O), lambda i: (i, 0, 0)),
        out_shape=jax.ShapeDtypeStruct((B, H, O), jnp.float32),
        scratch_shapes=[
            pltpu.VMEM((RB, E), jnp.float32),
            pltpu.SemaphoreType.DMA,
        ],
    )(xhbm, wt, b2)


def kernel(category, table, W, b):
    idx2d = category.astype(jnp.int32).reshape(N // GCHUNK, GCHUNK)
    gathered = _sc_gather(idx2d, table)
    return _tc_linear_relu(gathered, W.T, b.reshape(1, O))


# speedup vs baseline: 1.3604x; 1.3604x over previous
import functools

import jax
import jax.numpy as jnp
from jax import lax
from jax.experimental import pallas as pl
from jax.experimental.pallas import tpu as pltpu
from jax.experimental.pallas import tpu_sc as plsc

B, H, E, O = 16384, 50, 32, 64
N = B * H
NC, NS = 2, 16
NW = NC * NS
PER_W = N // NW
GCHUNK = 128
CHUNK = 1024
NG = CHUNK // GCHUNK
NCHUNKS = PER_W // CHUNK
FOLD = 128 // E
NF = N // FOLD
OF = O * FOLD
CF = CHUNK // FOLD


def _sc_gather(idx2d, table):
    mesh = plsc.VectorSubcoreMesh(core_axis_name="c", subcore_axis_name="s")

    @functools.partial(
        pl.kernel,
        mesh=mesh,
        out_type=jax.ShapeDtypeStruct((NF, 128), jnp.float32),
        scratch_types=[
            pltpu.VMEM((NG, GCHUNK), jnp.int32),
            pltpu.VMEM((CHUNK, E), jnp.float32),
            pltpu.VMEM((CF, 128), jnp.float32),
            pltpu.SemaphoreType.DMA,
        ],
        compiler_params=pltpu.CompilerParams(use_tc_tiling_on_sc=False),
    )
    def k(idx_hbm, table_hbm, out_hbm, idx_v, rows_v, rows_f, sem):
        wid = lax.axis_index("s") * NC + lax.axis_index("c")
        base = wid * PER_W

        def body(i, carry):
            off = pl.multiple_of(base + i * CHUNK, CHUNK)
            pltpu.sync_copy(idx_hbm.at[pl.ds(pl.multiple_of(off // GCHUNK, NG), NG)], idx_v)
            copies = [
                pltpu.async_copy(
                    table_hbm.at[idx_v.at[j]],
                    rows_v.at[pl.ds(j * GCHUNK, GCHUNK)],
                    sem,
                )
                for j in range(NG)
            ]
            for cp in copies:
                cp.wait()

            def fold_body(r, c2):
                for u in range(8):
                    v = rows_v[FOLD * r + u // 2, pl.ds(16 * (u % 2), 16)]
                    rows_f[r, pl.ds(16 * u, 16)] = v
                return c2

            lax.fori_loop(0, CF, fold_body, 0)

            pltpu.sync_copy(
                rows_f,
                out_hbm.at[pl.ds(pl.multiple_of(off // FOLD, CF), CF)],
            )
            return carry

        lax.fori_loop(0, NCHUNKS, body, 0)

    return k(idx2d, table)


def _tc_linear_relu(xf, wd, bf):
    blk = 1024

    def body(x_ref, w_ref, b_ref, o_ref):
        acc = jnp.dot(x_ref[...], w_ref[...], preferred_element_type=jnp.float32)
        o_ref[...] = jnp.maximum(acc + b_ref[...], 0.0)

    return pl.pallas_call(
        body,
        grid=(NF // blk,),
        in_specs=[
            pl.BlockSpec((blk, 128), lambda i: (i, 0)),
            pl.BlockSpec((128, OF), lambda i: (0, 0)),
            pl.BlockSpec((1, OF), lambda i: (0, 0)),
        ],
        out_specs=pl.BlockSpec((blk, OF), lambda i: (i, 0)),
        out_shape=jax.ShapeDtypeStruct((NF, OF), jnp.float32),
    )(xf, wd, bf)


def kernel(category, table, W, b):
    idx2d = category.astype(jnp.int32).reshape(N // GCHUNK, GCHUNK)
    xf = _sc_gather(idx2d, table)
    wd = jnp.kron(jnp.eye(FOLD, dtype=jnp.float32), W.T)
    bf = jnp.tile(b, FOLD).reshape(1, OF)
    out = _tc_linear_relu(xf, wd, bf)
    return out.reshape(B, H, O)


# speedup vs baseline: 1.4148x; 1.0400x over previous
import functools

import jax
import jax.numpy as jnp
from jax import lax
from jax.experimental import pallas as pl
from jax.experimental.pallas import tpu as pltpu
from jax.experimental.pallas import tpu_sc as plsc

B, H, E, O = 16384, 50, 32, 64
N = B * H
NC, NS = 2, 16
NW = NC * NS
PER_W = N // NW
GCHUNK = 128
CHUNK = 1024
NG = CHUNK // GCHUNK
NCHUNKS = PER_W // CHUNK
FOLD = 128 // E
NF = N // FOLD
OF = O * FOLD
CF = CHUNK // FOLD


def _sc_gather(idx2d, table):
    mesh = plsc.VectorSubcoreMesh(core_axis_name="c", subcore_axis_name="s")

    @functools.partial(
        pl.kernel,
        mesh=mesh,
        out_type=jax.ShapeDtypeStruct((NF, 128), jnp.float32),
        scratch_types=[
            pltpu.VMEM((NG, GCHUNK), jnp.int32),
            pltpu.VMEM((CHUNK, E), jnp.float32),
            pltpu.VMEM((CF, 128), jnp.float32),
            pltpu.SemaphoreType.DMA,
        ],
        compiler_params=pltpu.CompilerParams(use_tc_tiling_on_sc=False),
    )
    def k(idx_hbm, table_hbm, out_hbm, idx_v, rows_v, rows_f, sem):
        wid = lax.axis_index("s") * NC + lax.axis_index("c")
        base = wid * PER_W

        def body(i, carry):
            off = pl.multiple_of(base + i * CHUNK, CHUNK)
            pltpu.sync_copy(idx_hbm.at[pl.ds(pl.multiple_of(off // GCHUNK, NG), NG)], idx_v)
            copies = [
                pltpu.async_copy(
                    table_hbm.at[idx_v.at[j]],
                    rows_v.at[pl.ds(j * GCHUNK, GCHUNK)],
                    sem,
                )
                for j in range(NG)
            ]
            for cp in copies:
                cp.wait()

            def fold_body(r, c2):
                for u in range(8):
                    v = rows_v[FOLD * r + u // 2, pl.ds(16 * (u % 2), 16)]
                    rows_f[r, pl.ds(16 * u, 16)] = v
                return c2

            lax.fori_loop(0, CF, fold_body, 0)

            pltpu.sync_copy(
                rows_f,
                out_hbm.at[pl.ds(pl.multiple_of(off // FOLD, CF), CF)],
            )
            return carry

        lax.fori_loop(0, NCHUNKS, body, 0)

    return k(idx2d, table)


def _tc_linear_relu(xf, wd, bf):
    blk = 2048
    nblk = NF // blk

    def body(x_hbm, w_ref, b_ref, o_ref, xv, sem):
        i = pl.program_id(0)

        @pl.when(i == 0)
        def _():
            pltpu.make_async_copy(x_hbm.at[pl.ds(0, blk)], xv.at[0], sem).start()

        @pl.when(i + 1 < nblk)
        def _():
            pltpu.make_async_copy(
                x_hbm.at[pl.ds((i + 1) * blk, blk)], xv.at[(i + 1) % 2], sem
            ).start()

        pltpu.make_async_copy(x_hbm.at[pl.ds(i * blk, blk)], xv.at[i % 2], sem).wait()
        acc = jnp.dot(xv[i % 2], w_ref[...], preferred_element_type=jnp.float32)
        o_ref[...] = jnp.maximum(acc + b_ref[...], 0.0)

    return pl.pallas_call(
        body,
        grid=(nblk,),
        in_specs=[
            pl.BlockSpec(memory_space=pltpu.MemorySpace.HBM),
            pl.BlockSpec((128, OF), lambda i: (0, 0)),
            pl.BlockSpec((1, OF), lambda i: (0, 0)),
        ],
        out_specs=pl.BlockSpec((blk, OF), lambda i: (i, 0)),
        out_shape=jax.ShapeDtypeStruct((NF, OF), jnp.float32),
        scratch_shapes=[
            pltpu.VMEM((2, blk, 128), jnp.float32),
            pltpu.SemaphoreType.DMA,
        ],
    )(xf, wd, bf)


def kernel(category, table, W, b):
    idx2d = category.astype(jnp.int32).reshape(N // GCHUNK, GCHUNK)
    xf = _sc_gather(idx2d, table)
    wd = jnp.kron(jnp.eye(FOLD, dtype=jnp.float32), W.T)
    bf = jnp.tile(b, FOLD).reshape(1, OF)
    out = _tc_linear_relu(xf, wd, bf)
    return out.reshape(B, H, O)
